# baseline (speedup 1.0000x reference)
import math

import jax
import jax.numpy as jnp
from jax import lax
from jax.experimental import pallas as pl
from jax.experimental.pallas import tpu as pltpu
from jax.experimental.pallas import tpu_sc as plsc

VOCAB = 1000000
D = 64
ROWS = 16384
COLS = 200
B = ROWS * COLS

NC = 2
NS = 16
NW = NC * NS
PER_W = B // NW
CL = 800
NCH = PER_W // CL
SCALE = math.sqrt(D)

_mesh = plsc.VectorSubcoreMesh(
    core_axis_name="c", subcore_axis_name="s", num_cores=NC, num_subcores=NS
)


def _body(table_hbm, x_hbm, out_hbm, idx0, idx1, g0, g1,
          isem0, isem1, gsem0, gsem1, ssem0, ssem1):
    wid = lax.axis_index("s") * NC + lax.axis_index("c")
    base = wid * PER_W
    idx = [idx0, idx1]
    gbuf = [g0, g1]
    isem = [isem0, isem1]
    gsem = [gsem0, gsem1]
    ssem = [ssem0, ssem1]

    def start_idx(ch, b):
        pltpu.async_copy(
            x_hbm.at[pl.ds(base + ch * CL, CL)], idx[b], isem[b]
        )

    def wait_idx(b):
        pltpu.make_async_copy(
            x_hbm.at[pl.ds(0, CL)], idx[b], isem[b]
        ).wait()

    def start_gather(b):
        pltpu.async_copy(table_hbm.at[idx[b]], gbuf[b], gsem[b])

    def wait_gather(b):
        pltpu.make_async_copy(
            table_hbm.at[idx[b]], gbuf[b], gsem[b]
        ).wait()

    def start_scatter(ch, b):
        o0 = base + ch * CL
        pltpu.async_copy(
            gbuf[b], out_hbm.at[pl.ds(o0, CL), pl.ds(0, D)], ssem[b]
        )

    def wait_scatter(b):
        pltpu.make_async_copy(
            gbuf[b], out_hbm.at[pl.ds(0, CL), pl.ds(0, D)], ssem[b]
        ).wait()

    start_idx(0, 0)
    start_idx(1, 1)
    wait_idx(0)
    start_gather(0)

    @pl.loop(0, NCH, step=2)
    def _chunks(g):
        for b in range(2):
            ch = g + b
            nb = (b + 1) % 2
            nxt = ch + 1

            @pl.when(nxt < NCH)
            def _():
                wait_idx(nb)

                @pl.when(nxt >= 2)
                def _():
                    wait_scatter(nb)

                start_gather(nb)

            wait_gather(b)

            @pl.when(ch + 2 < NCH)
            def _():
                start_idx(ch + 2, b)

            @pl.loop(0, CL, unroll=8)
            def _scale(r):
                for k in range(D // 16):
                    sl = pl.ds(k * 16, 16)
                    gbuf[b][r, sl] = gbuf[b][r, sl] * SCALE

            start_scatter(ch, b)

    for b in range(2):
        wait_scatter(b)


_lookup = pl.kernel(
    _body,
    out_type=jax.ShapeDtypeStruct((B, 2 * D), jnp.float32),
    mesh=_mesh,
    scratch_types=[
        pltpu.VMEM((CL,), jnp.int32),
        pltpu.VMEM((CL,), jnp.int32),
        pltpu.VMEM((CL, D), jnp.float32),
        pltpu.VMEM((CL, D), jnp.float32),
        pltpu.SemaphoreType.DMA,
        pltpu.SemaphoreType.DMA,
        pltpu.SemaphoreType.DMA,
        pltpu.SemaphoreType.DMA,
        pltpu.SemaphoreType.DMA,
        pltpu.SemaphoreType.DMA,
    ],
    compiler_params=pltpu.CompilerParams(use_tc_tiling_on_sc=False),
)


@jax.jit
def kernel(x, table):
    out = _lookup(table, x.reshape(B))
    return out[:, :D].reshape(ROWS, COLS, D)


# speedup vs baseline: 1.0072x; 1.0072x over previous
import math

import jax
import jax.numpy as jnp
from jax import lax
from jax.experimental import pallas as pl
from jax.experimental.pallas import tpu as pltpu
from jax.experimental.pallas import tpu_sc as plsc

VOCAB = 1000000
D = 64
ROWS = 16384
COLS = 200
B = ROWS * COLS

NC = 2
NS = 16
NW = NC * NS
PER_W = B // NW
CL = 400
NCH = PER_W // CL
NBUF = 4
SCALE = math.sqrt(D)

_mesh = plsc.VectorSubcoreMesh(
    core_axis_name="c", subcore_axis_name="s", num_cores=NC, num_subcores=NS
)


def _body(table_hbm, x_hbm, out_hbm, *refs):
    idx = list(refs[0:NBUF])
    gbuf = list(refs[NBUF:2 * NBUF])
    isem = list(refs[2 * NBUF:3 * NBUF])
    gsem = list(refs[3 * NBUF:4 * NBUF])
    ssem = list(refs[4 * NBUF:5 * NBUF])

    wid = lax.axis_index("s") * NC + lax.axis_index("c")
    base = wid * PER_W

    def start_idx(ch, b):
        pltpu.async_copy(
            x_hbm.at[pl.ds(base + ch * CL, CL)], idx[b], isem[b]
        )

    def wait_idx(b):
        pltpu.make_async_copy(
            x_hbm.at[pl.ds(0, CL)], idx[b], isem[b]
        ).wait()

    def start_gather(b):
        pltpu.async_copy(table_hbm.at[idx[b]], gbuf[b], gsem[b])

    def wait_gather(b):
        pltpu.make_async_copy(
            table_hbm.at[idx[b]], gbuf[b], gsem[b]
        ).wait()

    def start_scatter(ch, b):
        o0 = base + ch * CL
        pltpu.async_copy(
            gbuf[b], out_hbm.at[pl.ds(o0, CL), pl.ds(0, D)], ssem[b]
        )

    def wait_scatter(b):
        pltpu.make_async_copy(
            gbuf[b], out_hbm.at[pl.ds(0, CL), pl.ds(0, D)], ssem[b]
        ).wait()

    for b in range(NBUF):
        start_idx(b, b)
    for b in range(2):
        wait_idx(b)
        start_gather(b)

    @pl.loop(0, NCH, step=NBUF)
    def _chunks(g):
        for b in range(NBUF):
            ch = g + b
            nxt = ch + 2
            nb = (b + 2) % NBUF

            @pl.when(nxt < NCH)
            def _():
                wait_idx(nb)

                @pl.when(nxt >= NBUF)
                def _():
                    wait_scatter(nb)

                start_gather(nb)

            wait_gather(b)

            @pl.when(ch + NBUF < NCH)
            def _():
                start_idx(ch + NBUF, b)

            @pl.loop(0, CL, unroll=8)
            def _scale(r):
                for k in range(D // 16):
                    sl = pl.ds(k * 16, 16)
                    gbuf[b][r, sl] = gbuf[b][r, sl] * SCALE

            start_scatter(ch, b)

    for b in range(NBUF):
        wait_scatter(b)


_lookup = pl.kernel(
    _body,
    out_type=jax.ShapeDtypeStruct((B, 2 * D), jnp.float32),
    mesh=_mesh,
    scratch_types=(
        [pltpu.VMEM((CL,), jnp.int32)] * NBUF
        + [pltpu.VMEM((CL, D), jnp.float32)] * NBUF
        + [pltpu.SemaphoreType.DMA] * (3 * NBUF)
    ),
    compiler_params=pltpu.CompilerParams(use_tc_tiling_on_sc=False),
)


@jax.jit
def kernel(x, table):
    out = _lookup(table, x.reshape(B))
    return out[:, :D].reshape(ROWS, COLS, D)
